# baseline (speedup 1.0000x reference)
import jax
import jax.numpy as jnp
from jax.experimental import pallas as pl
from jax.experimental.pallas import tpu as pltpu

E = 8
R = 8
SCALE = 1.0

TMA = 512
TM = 256


def _routing_kernel(x_ref, wg_ref, eb_ref, acat_ref, u_ref, xbf_ref):
    xb = x_ref[...]
    g = jax.lax.dot_general(xb, wg_ref[...], (((1,), (1,)), ((), ())),
                            preferred_element_type=jnp.float32)
    g = g + eb_ref[...]
    probs = jax.nn.softmax(g, axis=-1)
    top1 = jnp.argmax(probs, axis=-1)[:, None]
    u = jnp.dot(xb, acat_ref[...],
                preferred_element_type=jnp.float32)
    lane = jax.lax.broadcasted_iota(jnp.int32, (TMA, E * R), 1) // R
    mask = (lane == top1).astype(jnp.float32)
    u_ref[...] = u * (mask * SCALE)
    xbf_ref[...] = xb.astype(jnp.bfloat16)


def _main_kernel(xbf_ref, wut_ref, bu_ref, u_ref, bcat_ref, out_ref):
    base = jnp.dot(xbf_ref[...], wut_ref[...],
                   preferred_element_type=jnp.float32)
    delta = jnp.dot(u_ref[...], bcat_ref[...],
                    preferred_element_type=jnp.float32)
    out_ref[...] = base + bu_ref[...] + delta


def kernel(x, W_gate, expert_bias, W_up, b_up, A, B):
    Bb, T, H = x.shape
    NT = Bb * T
    x_flat = x.reshape(NT, H)
    W_upT = W_up.T.astype(jnp.bfloat16)
    A_cat = A.transpose(1, 0, 2).reshape(H, E * R)
    B_cat = B.reshape(E * R, H)
    eb = expert_bias.reshape(1, E)
    bu = b_up.reshape(1, H)

    u_masked, x_bf = pl.pallas_call(
        _routing_kernel,
        grid=(NT // TMA,),
        in_specs=[
            pl.BlockSpec((TMA, H), lambda t: (t, 0)),
            pl.BlockSpec((E, H), lambda t: (0, 0)),
            pl.BlockSpec((1, E), lambda t: (0, 0)),
            pl.BlockSpec((H, E * R), lambda t: (0, 0)),
        ],
        out_specs=[
            pl.BlockSpec((TMA, E * R), lambda t: (t, 0)),
            pl.BlockSpec((TMA, H), lambda t: (t, 0)),
        ],
        out_shape=[
            jax.ShapeDtypeStruct((NT, E * R), jnp.float32),
            jax.ShapeDtypeStruct((NT, H), jnp.bfloat16),
        ],
    )(x_flat, W_gate, eb, A_cat)

    out = pl.pallas_call(
        _main_kernel,
        grid=(NT // TM,),
        in_specs=[
            pl.BlockSpec((TM, H), lambda t: (t, 0)),
            pl.BlockSpec((H, H), lambda t: (0, 0)),
            pl.BlockSpec((1, H), lambda t: (0, 0)),
            pl.BlockSpec((TM, E * R), lambda t: (t, 0)),
            pl.BlockSpec((E * R, H), lambda t: (0, 0)),
        ],
        out_specs=pl.BlockSpec((TM, H), lambda t: (t, 0)),
        out_shape=jax.ShapeDtypeStruct((NT, H), jnp.float32),
    )(x_bf, W_upT, bu, u_masked, B_cat)
    return out.reshape(Bb, T, H)


# speedup vs baseline: 1.1490x; 1.1490x over previous
import jax
import jax.numpy as jnp
from jax.experimental import pallas as pl
from jax.experimental.pallas import tpu as pltpu

E = 8
R = 8
SCALE = 1.0

TM = 256


def _moe_lora_kernel(x_ref, wg_ref, eb_ref, wu_ref, bu_ref, acat_ref,
                     bcat_ref, out_ref):
    xb = x_ref[...]
    g = jax.lax.dot_general(xb, wg_ref[...], (((1,), (1,)), ((), ())),
                            preferred_element_type=jnp.float32)
    g = g + eb_ref[...]
    top1 = jnp.argmax(g, axis=-1)[:, None]
    u = jnp.dot(xb, acat_ref[...],
                preferred_element_type=jnp.float32)
    lane = jax.lax.broadcasted_iota(jnp.int32, (TM, E * R), 1) // R
    mask = (lane == top1).astype(jnp.float32)
    u_masked = u * (mask * SCALE)
    base = jax.lax.dot_general(xb.astype(jnp.bfloat16), wu_ref[...],
                               (((1,), (1,)), ((), ())),
                               preferred_element_type=jnp.float32)
    delta = jnp.dot(u_masked, bcat_ref[...],
                    preferred_element_type=jnp.float32)
    out_ref[...] = base + bu_ref[...] + delta


def kernel(x, W_gate, expert_bias, W_up, b_up, A, B):
    Bb, T, H = x.shape
    NT = Bb * T
    x_flat = x.reshape(NT, H)
    W_bf = W_up.astype(jnp.bfloat16)
    A_cat = A.transpose(1, 0, 2).reshape(H, E * R)
    B_cat = B.reshape(E * R, H)
    eb = expert_bias.reshape(1, E)
    bu = b_up.reshape(1, H)
    TB = NT // TM

    out = pl.pallas_call(
        _moe_lora_kernel,
        grid=(TB,),
        in_specs=[
            pl.BlockSpec((TM, H), lambda t: (t, 0)),
            pl.BlockSpec((E, H), lambda t: (0, 0)),
            pl.BlockSpec((1, E), lambda t: (0, 0)),
            pl.BlockSpec((H, H), lambda t: (0, 0)),
            pl.BlockSpec((1, H), lambda t: (0, 0)),
            pl.BlockSpec((H, E * R), lambda t: (0, 0)),
            pl.BlockSpec((E * R, H), lambda t: (0, 0)),
        ],
        out_specs=pl.BlockSpec((TM, H), lambda t: (t, 0)),
        out_shape=jax.ShapeDtypeStruct((NT, H), jnp.float32),
    )(x_flat, W_gate, eb, W_bf, bu, A_cat, B_cat)
    return out.reshape(Bb, T, H)
